# baseline (speedup 1.0000x reference)
import jax
import jax.numpy as jnp
from jax.experimental import pallas as pl

_SIGMA_X = 1.0
_SIGMA_Y = 1.0


def _preprocess_kernel(x_ref, o_ref):
    xv = x_ref[...]
    B, _, H, W = xv.shape
    m = (xv[:, 2, :, :] > 0).astype(jnp.float32)
    mf = m.reshape(B * H, W)

    rj = jax.lax.broadcasted_iota(jnp.int32, (W, W), 0)
    cj = jax.lax.broadcasted_iota(jnp.int32, (W, W), 1)
    dy = (rj - cj).astype(jnp.float32)
    ky = jnp.exp(-(dy * dy) / (2.0 * _SIGMA_Y * _SIGMA_Y))

    rb = jax.lax.broadcasted_iota(jnp.int32, (B * H, B * H), 0)
    cb = jax.lax.broadcasted_iota(jnp.int32, (B * H, B * H), 1)
    dx = ((rb % H) - (cb % H)).astype(jnp.float32)
    kxb = jnp.where(
        (rb // H) == (cb // H),
        jnp.exp(-(dx * dx) / (2.0 * _SIGMA_X * _SIGMA_X)),
        0.0,
    )

    a = jnp.dot(mf, ky, precision=jax.lax.Precision.HIGHEST)
    heat = jnp.dot(kxb, a, precision=jax.lax.Precision.HIGHEST)
    heat3 = heat.reshape(B, H, W)
    mx = jnp.max(heat3, axis=(1, 2), keepdims=True)
    normed = heat3 / jnp.where(mx == 0.0, 1.0, mx)

    keep = jnp.sum(mf) > 0.0
    o_ref[...] = xv
    o_ref[:, 2, :, :] = jnp.where(keep, normed, xv[:, 2, :, :])


@jax.jit
def kernel(x):
    return pl.pallas_call(
        _preprocess_kernel,
        out_shape=jax.ShapeDtypeStruct(x.shape, x.dtype),
    )(x)


# speedup vs baseline: 1.1221x; 1.1221x over previous
import jax
import jax.numpy as jnp
from jax.experimental import pallas as pl

_SIGMA_X = 1.0
_SIGMA_Y = 1.0


def _preprocess_kernel(x_ref, o_ref):
    xv = x_ref[...]
    B, _, H, W = xv.shape
    m = (xv[:, 2, :, :] > 0).astype(jnp.float32)
    mf = m.reshape(B * H, W)

    rj = jax.lax.broadcasted_iota(jnp.int32, (W, W), 0)
    cj = jax.lax.broadcasted_iota(jnp.int32, (W, W), 1)
    dy = (rj - cj).astype(jnp.float32)
    ky = jnp.exp(-(dy * dy) / (2.0 * _SIGMA_Y * _SIGMA_Y))

    ri = jax.lax.broadcasted_iota(jnp.int32, (H, H), 0)
    ci = jax.lax.broadcasted_iota(jnp.int32, (H, H), 1)
    dx = (ri - ci).astype(jnp.float32)
    kx = jnp.exp(-(dx * dx) / (2.0 * _SIGMA_X * _SIGMA_X))

    t = jnp.concatenate(
        [jnp.dot(kx, m[b], precision=jax.lax.Precision.HIGHEST) for b in range(B)],
        axis=0,
    )
    heat = jnp.dot(t, ky, precision=jax.lax.Precision.HIGHEST)
    heat3 = heat.reshape(B, H, W)
    mx = jnp.max(heat3, axis=(1, 2), keepdims=True)
    normed = heat3 / jnp.where(mx == 0.0, 1.0, mx)

    keep = jnp.sum(mf) > 0.0
    o_ref[...] = xv
    o_ref[:, 2, :, :] = jnp.where(keep, normed, xv[:, 2, :, :])


@jax.jit
def kernel(x):
    return pl.pallas_call(
        _preprocess_kernel,
        out_shape=jax.ShapeDtypeStruct(x.shape, x.dtype),
    )(x)
